# Initial kernel scaffold; baseline (speedup 1.0000x reference)
import jax
import jax.numpy as jnp
from jax.experimental import pallas as pl


def kernel(x, batch, params):
    raise NotImplementedError("write your pallas kernel here")


# speedup vs baseline: 4.0456x; 4.0456x over previous
import jax
import jax.numpy as jnp
from jax.experimental import pallas as pl

_N = 10000
_G = 5
_K = 4
_QT = 200
_HI = jax.lax.Precision.HIGHEST


def _dot(a, b):
    return jax.lax.dot_general(a, b, (((1,), (0,)), ((), ())),
                               precision=_HI, preferred_element_type=jnp.float32)


def _dotL(a, b):
    return jax.lax.dot_general(a, b, (((1,), (0,)), ((), ())),
                               preferred_element_type=jnp.float32)


def _dot0(a, b):
    return jax.lax.dot_general(a, b, (((0,), (0,)), ((), ())),
                               precision=_HI, preferred_element_type=jnp.float32)


def _bf16dot3(a_bf, b_hi, b_mid, b_lo):
    o = jax.lax.dot_general(a_bf, b_hi, (((1,), (0,)), ((), ())),
                            preferred_element_type=jnp.float32)
    o = o + jax.lax.dot_general(a_bf, b_mid, (((1,), (0,)), ((), ())),
                                preferred_element_type=jnp.float32)
    o = o + jax.lax.dot_general(a_bf, b_lo, (((1,), (0,)), ((), ())),
                                preferred_element_type=jnp.float32)
    return o


def _bn(y, g, b):
    m = jnp.mean(y, axis=0, keepdims=True)
    v = jnp.mean((y - m) ** 2, axis=0, keepdims=True)
    return (y - m) / jnp.sqrt(v + 1e-5) * g + b


def _pre_body(x_ref, bcol_ref,
              w0, b0, g0, be0, w1, b1, g1, be1, w2, b2, g2, be2,
              wh, bh, ws, bs,
              x1_ref, h_ref, s_ref):
    x = x_ref[...]
    bat = bcol_ref[...]
    gid = jax.lax.broadcasted_iota(jnp.int32, (_N, _G), 1)
    onehot = (bat == gid).astype(jnp.float32)
    cnt = _dot0(onehot, jnp.ones((_N, 1), jnp.float32))
    sums = _dot0(onehot, x)
    mean = sums / jnp.maximum(cnt, 1.0)
    mns, mxs = [], []
    for g in range(_G):
        mask = bat == g
        mns.append(jnp.min(jnp.where(mask, x, jnp.inf), axis=0, keepdims=True))
        mxs.append(jnp.max(jnp.where(mask, x, -jnp.inf), axis=0, keepdims=True))
    mn = jnp.concatenate(mns, axis=0)
    mx = jnp.concatenate(mxs, axis=0)
    mmm = jnp.concatenate([mean, mn, mx], axis=1)
    gath = _dot(onehot, mmm)
    xx = jnp.concatenate([gath, x], axis=1)
    for (w, b, g, be) in ((w0, b0, g0, be0), (w1, b1, g1, be1), (w2, b2, g2, be2)):
        xx = _bn(jax.nn.relu(_dotL(xx, w[...]) + b[...]), g[...], be[...])
    x1_ref[...] = xx
    h_ref[...] = _dotL(xx, wh[...]) + bh[...]
    s_ref[...] = _dotL(xx, ws[...]) + bs[...]


def _knn_body(qs_ref, qb_ref, st_ref, brow_ref, s_ref, h_ref, agg_ref):
    qs = qs_ref[...]
    qb = qb_ref[...]
    st = st_ref[...]
    brow = brow_ref[...]
    s = s_ref[...]
    h = h_ref[...]
    sh = jnp.concatenate([s, h], axis=1)
    sh_hi = sh.astype(jnp.bfloat16)
    r1 = sh - sh_hi.astype(jnp.float32)
    sh_mid = r1.astype(jnp.bfloat16)
    sh_lo = (r1 - sh_mid.astype(jnp.float32)).astype(jnp.bfloat16)
    p = [(qs[:, c:c + 1] - st[c:c + 1, :]) ** 2 for c in range(4)]
    d = (p[0] + p[1]) + (p[2] + p[3])
    d = jnp.where(qb == brow, d, jnp.inf)
    iota = jax.lax.broadcasted_iota(jnp.int32, (_QT, _N), 1)
    acc_s = acc_m = None
    for t in range(_K):
        m = jnp.min(d, axis=1, keepdims=True)
        jstar = jnp.min(jnp.where(d <= m, iota, _N), axis=1, keepdims=True)
        onehot_b = iota == jstar
        onehot = onehot_b.astype(jnp.bfloat16)
        shv = _bf16dot3(onehot, sh_hi, sh_mid, sh_lo)
        sv = shv[:, :4]
        hv = shv[:, 4:]
        q2 = [(qs[:, c:c + 1] - sv[:, c:c + 1]) ** 2 for c in range(4)]
        d2 = (q2[0] + q2[1]) + (q2[2] + q2[3])
        w = jnp.exp(-10.0 * d2)
        msg = hv * w
        acc_s = msg if acc_s is None else acc_s + msg
        acc_m = msg if acc_m is None else jnp.maximum(acc_m, msg)
        if t + 1 < _K:
            d = jnp.where(onehot_b, jnp.inf, d)
    agg_ref[...] = jnp.concatenate([acc_s * (1.0 / _K), acc_m], axis=1)


def _knn_call(s, h, bcol, brow):
    st = s.T
    return pl.pallas_call(
        _knn_body,
        grid=(_N // _QT,),
        in_specs=[
            pl.BlockSpec((_QT, 4), lambda i: (i, 0)),
            pl.BlockSpec((_QT, 1), lambda i: (i, 0)),
            pl.BlockSpec((4, _N), lambda i: (0, 0)),
            pl.BlockSpec((1, _N), lambda i: (0, 0)),
            pl.BlockSpec((_N, 4), lambda i: (0, 0)),
            pl.BlockSpec((_N, 22), lambda i: (0, 0)),
        ],
        out_specs=pl.BlockSpec((_QT, 44), lambda i: (i, 0)),
        out_shape=jax.ShapeDtypeStruct((_N, 44), jnp.float32),
    )(s, bcol, st, brow, s, h)


def _block_mlp(x, agg, wo, bo, g1, be1, wf1, bf1, g2, be2, wf2, bf2):
    y = _dotL(jnp.concatenate([x, agg], axis=1), wo[...]) + bo[...]
    y = _bn(y, g1[...], be1[...])
    y = jax.nn.relu(_dotL(y, wf1[...]) + bf1[...])
    y = _bn(y, g2[...], be2[...])
    return _dotL(y, wf2[...]) + bf2[...]


def _mlp_body(x_ref, agg_ref, wo, bo, g1, be1, wf1, bf1, g2, be2, wf2, bf2,
              xo_ref):
    xo_ref[...] = _block_mlp(x_ref[...], agg_ref[...], wo, bo, g1, be1,
                             wf1, bf1, g2, be2, wf2, bf2)


def _mlp_hs_body(x_ref, agg_ref, wo, bo, g1, be1, wf1, bf1, g2, be2, wf2, bf2,
                 wh, bh, ws, bs, xo_ref, h_ref, s_ref):
    y = _block_mlp(x_ref[...], agg_ref[...], wo, bo, g1, be1,
                   wf1, bf1, g2, be2, wf2, bf2)
    xo_ref[...] = y
    h_ref[...] = _dotL(y, wh[...]) + bh[...]
    s_ref[...] = _dotL(y, ws[...]) + bs[...]


def _post_body(x0, x1, x2, x3,
               pw0, pb0, pg0, pbe0, pw1, pb1, pg1, pbe1,
               pw2, pb2, pg2, pbe2, pw3, pb3, pg3, pbe3,
               ow1, ob1, ow2, ob2, ow3, ob3, y_ref):
    y = jnp.concatenate([x0[...], x1[...], x2[...], x3[...]], axis=1)
    for (w, b, g, be) in ((pw0, pb0, pg0, pbe0), (pw1, pb1, pg1, pbe1),
                          (pw2, pb2, pg2, pbe2), (pw3, pb3, pg3, pbe3)):
        y = _bn(jax.nn.relu(_dotL(y, w[...]) + b[...]), g[...], be[...])
    y = jax.nn.relu(_dotL(y, ow1[...]) + ob1[...])
    y = jax.nn.relu(_dotL(y, ow2[...]) + ob2[...])
    y_ref[...] = jax.nn.sigmoid(_dotL(y, ow3[...]) + ob3[...])


def kernel(x, batch, params):
    bcol = batch.reshape(_N, 1).astype(jnp.int32)
    brow = batch.reshape(1, _N).astype(jnp.int32)

    def r2(a):
        return a.reshape(1, -1)

    blk = params["blocks"]
    pre_w = []
    for p in params["pre"]:
        pre_w += [p["lin"]["w"], r2(p["lin"]["b"]),
                  r2(p["bn"]["g"]), r2(p["bn"]["b"])]
    b0 = blk[0]
    xk, h, s = pl.pallas_call(
        _pre_body,
        out_shape=[
            jax.ShapeDtypeStruct((_N, 64), jnp.float32),
            jax.ShapeDtypeStruct((_N, 22), jnp.float32),
            jax.ShapeDtypeStruct((_N, 4), jnp.float32),
        ],
    )(x, bcol, *pre_w,
      b0["lin_h"]["w"], r2(b0["lin_h"]["b"]),
      b0["lin_s"]["w"], r2(b0["lin_s"]["b"]))

    outs = []
    for k in range(4):
        agg = _knn_call(s, h, bcol, brow)
        bk = blk[k]
        mlp_w = [bk["lin_out"]["w"], r2(bk["lin_out"]["b"]),
                 r2(bk["bn1"]["g"]), r2(bk["bn1"]["b"]),
                 bk["fc1"]["w"], r2(bk["fc1"]["b"]),
                 r2(bk["bn2"]["g"]), r2(bk["bn2"]["b"]),
                 bk["fc2"]["w"], r2(bk["fc2"]["b"])]
        if k < 3:
            nb = blk[k + 1]
            xk, h, s = pl.pallas_call(
                _mlp_hs_body,
                out_shape=[
                    jax.ShapeDtypeStruct((_N, 64), jnp.float32),
                    jax.ShapeDtypeStruct((_N, 22), jnp.float32),
                    jax.ShapeDtypeStruct((_N, 4), jnp.float32),
                ],
            )(xk, agg, *mlp_w,
              nb["lin_h"]["w"], r2(nb["lin_h"]["b"]),
              nb["lin_s"]["w"], r2(nb["lin_s"]["b"]))
        else:
            xk = pl.pallas_call(
                _mlp_body,
                out_shape=jax.ShapeDtypeStruct((_N, 64), jnp.float32),
            )(xk, agg, *mlp_w)
        outs.append(xk)

    post_w = []
    for p in params["post"]:
        post_w += [p["lin"]["w"], r2(p["lin"]["b"]),
                   r2(p["bn"]["g"]), r2(p["bn"]["b"])]
    y = pl.pallas_call(
        _post_body,
        out_shape=jax.ShapeDtypeStruct((_N, 1), jnp.float32),
    )(*outs, *post_w,
      params["out1"]["w"], r2(params["out1"]["b"]),
      params["out2"]["w"], r2(params["out2"]["b"]),
      params["out3"]["w"], r2(params["out3"]["b"]))
    return y


# speedup vs baseline: 11.4585x; 2.8323x over previous
import jax
import jax.numpy as jnp
from jax.experimental import pallas as pl
from jax.experimental.pallas import tpu as pltpu

_N = 10000
_G = 5
_K = 4
_QT = 256
_CH = 512
_NPQ = 10240
_NP = _NPQ + _CH
_NLOC = 10240
_HI = jax.lax.Precision.HIGHEST


def _dot(a, b):
    return jax.lax.dot_general(a, b, (((1,), (0,)), ((), ())),
                               precision=_HI, preferred_element_type=jnp.float32)


def _dotL(a, b):
    return jax.lax.dot_general(a, b, (((1,), (0,)), ((), ())),
                               preferred_element_type=jnp.float32)


def _dot0(a, b):
    return jax.lax.dot_general(a, b, (((0,), (0,)), ((), ())),
                               precision=_HI, preferred_element_type=jnp.float32)


def _dot0b(a, b):
    return jax.lax.dot_general(a, b, (((0,), (0,)), ((), ())),
                               preferred_element_type=jnp.float32)


def _bn(y, g, b):
    m = jnp.mean(y, axis=0, keepdims=True)
    v = jnp.mean((y - m) ** 2, axis=0, keepdims=True)
    return (y - m) / jnp.sqrt(v + 1e-5) * g + b


def _hsplit(h):
    hh = h.astype(jnp.bfloat16)
    hl = (h - hh.astype(jnp.float32)).astype(jnp.bfloat16)
    return hh, hl


def _pre_body(x_ref, bcol_ref,
              w0, b0, g0, be0, w1, b1, g1, be1, w2, b2, g2, be2,
              wh, bh, ws, bs,
              x1_ref, s_ref, hh_ref, hl_ref):
    x = x_ref[...]
    bat = bcol_ref[...]
    gid = jax.lax.broadcasted_iota(jnp.int32, (_N, _G), 1)
    onehot = (bat == gid).astype(jnp.float32)
    cnt = _dot0(onehot, jnp.ones((_N, 1), jnp.float32))
    sums = _dot0(onehot, x)
    mean = sums / jnp.maximum(cnt, 1.0)
    mns, mxs = [], []
    for g in range(_G):
        mask = bat == g
        mns.append(jnp.min(jnp.where(mask, x, jnp.inf), axis=0, keepdims=True))
        mxs.append(jnp.max(jnp.where(mask, x, -jnp.inf), axis=0, keepdims=True))
    mn = jnp.concatenate(mns, axis=0)
    mx = jnp.concatenate(mxs, axis=0)
    mmm = jnp.concatenate([mean, mn, mx], axis=1)
    gath = _dot(onehot, mmm)
    xx = jnp.concatenate([gath, x], axis=1)
    for (w, b, g, be) in ((w0, b0, g0, be0), (w1, b1, g1, be1), (w2, b2, g2, be2)):
        xx = _bn(jax.nn.relu(_dotL(xx, w[...]) + b[...]), g[...], be[...])
    x1_ref[...] = xx
    s_ref[...] = _dotL(xx, ws[...]) + bs[...]
    hh_ref[...], hl_ref[...] = _hsplit(_dotL(xx, wh[...]) + bh[...])


def _knn_body(bounds_ref, qs_ref, qb_ref, sk_ref, bk_ref, hh_ref, hl_ref,
              agg_ref, dsc_ref):
    i = pl.program_id(0)
    lo = pl.multiple_of(bounds_ref[i, 0], _CH)
    nch = bounds_ref[i, 1]
    qT = qs_ref[...].T
    qbT = qb_ref[...].T
    inf = jnp.float32(jnp.inf)
    bigi = jnp.int32(2 ** 30)

    def iota(c):
        return (jax.lax.broadcasted_iota(jnp.int32, (_CH, _QT), 0) + c * _CH)

    def upd(dch, c, m, j):
        lio = iota(c)
        mc = jnp.min(dch, axis=0, keepdims=True)
        jc = jnp.min(jnp.where(dch <= mc, lio, bigi), axis=0, keepdims=True)
        better = mc < m
        return jnp.where(better, mc, m), jnp.where(better, jc, j)

    def body0(c, carry):
        m, j = carry
        k0 = lo + c * _CH
        sp = sk_ref[pl.ds(k0, _CH), :]
        bp = bk_ref[pl.ds(k0, _CH), :]
        p = [(sp[:, t:t + 1] - qT[t:t + 1, :]) ** 2 for t in range(4)]
        dch = (p[0] + p[1]) + (p[2] + p[3])
        dch = jnp.where(bp == qbT, dch, inf)
        dsc_ref[pl.ds(c * _CH, _CH), :] = dch
        return upd(dch, c, m, j)

    init = (jnp.full((1, _QT), inf), jnp.full((1, _QT), bigi, jnp.int32))
    ms, js = [], []
    m, j = jax.lax.fori_loop(0, nch, body0, init)
    ms.append(m)
    js.append(j)
    for _ in range(1, _K):
        def bodyt(c, carry):
            m, j = carry
            dch = dsc_ref[pl.ds(c * _CH, _CH), :]
            lio = iota(c)
            excl = lio == js[0]
            for jp in js[1:]:
                excl |= lio == jp
            dch = jnp.where(excl, inf, dch)
            return upd(dch, c, m, j)
        m, j = jax.lax.fori_loop(0, nch, bodyt, init)
        ms.append(m)
        js.append(j)

    def bodyg(c, accs):
        k0 = lo + c * _CH
        hh = hh_ref[pl.ds(k0, _CH), :]
        hl = hl_ref[pl.ds(k0, _CH), :]
        lio = iota(c)
        out = []
        for jt, acc in zip(js, accs):
            oh = (lio == jt).astype(jnp.bfloat16)
            out.append(acc + _dot0b(oh, hh) + _dot0b(oh, hl))
        return tuple(out)

    zero = jnp.zeros((_QT, 22), jnp.float32)
    accs = jax.lax.fori_loop(0, nch, bodyg, (zero,) * _K)

    acc_s = acc_m = None
    for m, acc in zip(ms, accs):
        w = jnp.exp(-10.0 * m).T
        msg = acc * w
        acc_s = msg if acc_s is None else acc_s + msg
        acc_m = msg if acc_m is None else jnp.maximum(acc_m, msg)
    agg_ref[...] = jnp.concatenate([acc_s * (1.0 / _K), acc_m], axis=1)


def _knn_call(bounds, s_pad, bcol_pad, hh_pad, hl_pad):
    full4 = pl.BlockSpec((_NP, 4), lambda i: (0, 0))
    full1 = pl.BlockSpec((_NP, 1), lambda i: (0, 0))
    full22 = pl.BlockSpec((_NP, 22), lambda i: (0, 0))
    return pl.pallas_call(
        _knn_body,
        grid=(_NPQ // _QT,),
        in_specs=[
            pl.BlockSpec(memory_space=pltpu.SMEM),
            pl.BlockSpec((_QT, 4), lambda i: (i, 0)),
            pl.BlockSpec((_QT, 1), lambda i: (i, 0)),
            full4, full1, full22, full22,
        ],
        out_specs=pl.BlockSpec((_QT, 44), lambda i: (i, 0)),
        out_shape=jax.ShapeDtypeStruct((_NPQ, 44), jnp.float32),
        scratch_shapes=[pltpu.VMEM((_NLOC, _QT), jnp.float32)],
    )(bounds, s_pad, bcol_pad, s_pad, bcol_pad, hh_pad, hl_pad)


def _block_mlp(x, agg, wo, bo, g1, be1, wf1, bf1, g2, be2, wf2, bf2):
    y = _dotL(jnp.concatenate([x, agg], axis=1), wo[...]) + bo[...]
    y = _bn(y, g1[...], be1[...])
    y = jax.nn.relu(_dotL(y, wf1[...]) + bf1[...])
    y = _bn(y, g2[...], be2[...])
    return _dotL(y, wf2[...]) + bf2[...]


def _mlp_body(x_ref, agg_ref, wo, bo, g1, be1, wf1, bf1, g2, be2, wf2, bf2,
              xo_ref):
    xo_ref[...] = _block_mlp(x_ref[...], agg_ref[...], wo, bo, g1, be1,
                             wf1, bf1, g2, be2, wf2, bf2)


def _mlp_hs_body(x_ref, agg_ref, wo, bo, g1, be1, wf1, bf1, g2, be2, wf2, bf2,
                 wh, bh, ws, bs, xo_ref, s_ref, hh_ref, hl_ref):
    y = _block_mlp(x_ref[...], agg_ref[...], wo, bo, g1, be1,
                   wf1, bf1, g2, be2, wf2, bf2)
    xo_ref[...] = y
    s_ref[...] = _dotL(y, ws[...]) + bs[...]
    hh_ref[...], hl_ref[...] = _hsplit(_dotL(y, wh[...]) + bh[...])


def _post_body(x0, x1, x2, x3,
               pw0, pb0, pg0, pbe0, pw1, pb1, pg1, pbe1,
               pw2, pb2, pg2, pbe2, pw3, pb3, pg3, pbe3,
               ow1, ob1, ow2, ob2, ow3, ob3, y_ref):
    y = jnp.concatenate([x0[...], x1[...], x2[...], x3[...]], axis=1)
    for (w, b, g, be) in ((pw0, pb0, pg0, pbe0), (pw1, pb1, pg1, pbe1),
                          (pw2, pb2, pg2, pbe2), (pw3, pb3, pg3, pbe3)):
        y = _bn(jax.nn.relu(_dotL(y, w[...]) + b[...]), g[...], be[...])
    y = jax.nn.relu(_dotL(y, ow1[...]) + ob1[...])
    y = jax.nn.relu(_dotL(y, ow2[...]) + ob2[...])
    y_ref[...] = jax.nn.sigmoid(_dotL(y, ow3[...]) + ob3[...])


def _tile_bounds(b32):
    starts = jnp.searchsorted(b32, jnp.arange(_G + 1, dtype=jnp.int32))
    bq = jnp.concatenate(
        [b32, jnp.full((_NPQ - _N,), b32[-1], jnp.int32)])
    bfirst = bq[0::_QT]
    blast = bq[_QT - 1::_QT]
    lo = starts[bfirst].astype(jnp.int32)
    hi = starts[blast + 1].astype(jnp.int32)
    lo = (lo // _CH) * _CH
    nch = (hi - lo + _CH - 1) // _CH
    return jnp.stack([lo, nch], axis=1).astype(jnp.int32)


def kernel(x, batch, params):
    b32 = batch.astype(jnp.int32)
    bcol = b32.reshape(_N, 1)
    bounds = _tile_bounds(b32)
    bcol_pad = jnp.pad(bcol, ((0, _NP - _N), (0, 0)), constant_values=-1)

    def r2(a):
        return a.reshape(1, -1)

    def padk(a):
        return jnp.pad(a, ((0, _NP - _N), (0, 0)))

    blk = params["blocks"]
    pre_w = []
    for p in params["pre"]:
        pre_w += [p["lin"]["w"], r2(p["lin"]["b"]),
                  r2(p["bn"]["g"]), r2(p["bn"]["b"])]
    b0 = blk[0]
    xk, s, hh, hl = pl.pallas_call(
        _pre_body,
        out_shape=[
            jax.ShapeDtypeStruct((_N, 64), jnp.float32),
            jax.ShapeDtypeStruct((_N, 4), jnp.float32),
            jax.ShapeDtypeStruct((_N, 22), jnp.bfloat16),
            jax.ShapeDtypeStruct((_N, 22), jnp.bfloat16),
        ],
    )(x, bcol, *pre_w,
      b0["lin_h"]["w"], r2(b0["lin_h"]["b"]),
      b0["lin_s"]["w"], r2(b0["lin_s"]["b"]))

    outs = []
    for k in range(4):
        agg = _knn_call(bounds, padk(s), bcol_pad, padk(hh), padk(hl))[:_N]
        bk = blk[k]
        mlp_w = [bk["lin_out"]["w"], r2(bk["lin_out"]["b"]),
                 r2(bk["bn1"]["g"]), r2(bk["bn1"]["b"]),
                 bk["fc1"]["w"], r2(bk["fc1"]["b"]),
                 r2(bk["bn2"]["g"]), r2(bk["bn2"]["b"]),
                 bk["fc2"]["w"], r2(bk["fc2"]["b"])]
        if k < 3:
            nb = blk[k + 1]
            xk, s, hh, hl = pl.pallas_call(
                _mlp_hs_body,
                out_shape=[
                    jax.ShapeDtypeStruct((_N, 64), jnp.float32),
                    jax.ShapeDtypeStruct((_N, 4), jnp.float32),
                    jax.ShapeDtypeStruct((_N, 22), jnp.bfloat16),
                    jax.ShapeDtypeStruct((_N, 22), jnp.bfloat16),
                ],
            )(xk, agg, *mlp_w,
              nb["lin_h"]["w"], r2(nb["lin_h"]["b"]),
              nb["lin_s"]["w"], r2(nb["lin_s"]["b"]))
        else:
            xk = pl.pallas_call(
                _mlp_body,
                out_shape=jax.ShapeDtypeStruct((_N, 64), jnp.float32),
            )(xk, agg, *mlp_w)
        outs.append(xk)

    post_w = []
    for p in params["post"]:
        post_w += [p["lin"]["w"], r2(p["lin"]["b"]),
                   r2(p["bn"]["g"]), r2(p["bn"]["b"])]
    y = pl.pallas_call(
        _post_body,
        out_shape=jax.ShapeDtypeStruct((_N, 1), jnp.float32),
    )(*outs, *post_w,
      params["out1"]["w"], r2(params["out1"]["b"]),
      params["out2"]["w"], r2(params["out2"]["b"]),
      params["out3"]["w"], r2(params["out3"]["b"]))
    return y


# speedup vs baseline: 12.7047x; 1.1088x over previous
import jax
import jax.numpy as jnp
from jax.experimental import pallas as pl
from jax.experimental.pallas import tpu as pltpu

_N = 10000
_G = 5
_K = 4
_QT = 256
_CH = 512
_NPQ = 10240
_NP = _NPQ + _CH
_HI = jax.lax.Precision.HIGHEST


def _dot(a, b):
    return jax.lax.dot_general(a, b, (((1,), (0,)), ((), ())),
                               precision=_HI, preferred_element_type=jnp.float32)


def _dotL(a, b):
    return jax.lax.dot_general(a, b, (((1,), (0,)), ((), ())),
                               preferred_element_type=jnp.float32)


def _dot0(a, b):
    return jax.lax.dot_general(a, b, (((0,), (0,)), ((), ())),
                               precision=_HI, preferred_element_type=jnp.float32)


def _dot0b(a, b):
    return jax.lax.dot_general(a, b, (((0,), (0,)), ((), ())),
                               preferred_element_type=jnp.float32)


def _bn(y, g, b):
    m = jnp.mean(y, axis=0, keepdims=True)
    v = jnp.mean((y - m) ** 2, axis=0, keepdims=True)
    return (y - m) / jnp.sqrt(v + 1e-5) * g + b


def _hsplit(h):
    hh = h.astype(jnp.bfloat16)
    hl = (h - hh.astype(jnp.float32)).astype(jnp.bfloat16)
    return jnp.concatenate([hh, hl], axis=1)


def _pre_body(x_ref, bcol_ref,
              w0, b0, g0, be0, w1, b1, g1, be1, w2, b2, g2, be2,
              wh, bh, ws, bs,
              x1_ref, s_ref, hc_ref):
    x = x_ref[...]
    bat = bcol_ref[...]
    gid = jax.lax.broadcasted_iota(jnp.int32, (_N, _G), 1)
    onehot = (bat == gid).astype(jnp.float32)
    cnt = _dot0(onehot, jnp.ones((_N, 1), jnp.float32))
    sums = _dot0(onehot, x)
    mean = sums / jnp.maximum(cnt, 1.0)
    mns, mxs = [], []
    for g in range(_G):
        mask = bat == g
        mns.append(jnp.min(jnp.where(mask, x, jnp.inf), axis=0, keepdims=True))
        mxs.append(jnp.max(jnp.where(mask, x, -jnp.inf), axis=0, keepdims=True))
    mn = jnp.concatenate(mns, axis=0)
    mx = jnp.concatenate(mxs, axis=0)
    mmm = jnp.concatenate([mean, mn, mx], axis=1)
    gath = _dot(onehot, mmm)
    xx = jnp.concatenate([gath, x], axis=1)
    for (w, b, g, be) in ((w0, b0, g0, be0), (w1, b1, g1, be1), (w2, b2, g2, be2)):
        xx = _bn(jax.nn.relu(_dotL(xx, w[...]) + b[...]), g[...], be[...])
    x1_ref[...] = xx
    s_ref[...] = _dotL(xx, ws[...]) + bs[...]
    hc_ref[...] = _hsplit(_dotL(xx, wh[...]) + bh[...])


def _knn_body(bounds_ref, qs_ref, qb_ref, sk_ref, bk_ref, hc_ref, agg_ref):
    i = pl.program_id(0)
    lo = pl.multiple_of(bounds_ref[i, 0], _CH)
    nch = bounds_ref[i, 1]
    qT = qs_ref[...].T
    qbT = qb_ref[...].T
    inf = jnp.float32(jnp.inf)
    bigi = jnp.int32(2 ** 30)

    def iota(c):
        return (jax.lax.broadcasted_iota(jnp.int32, (_CH, _QT), 0) + c * _CH)

    def body0(c, carry):
        ms, js = carry
        k0 = lo + c * _CH
        sp = sk_ref[pl.ds(k0, _CH), :]
        bp = bk_ref[pl.ds(k0, _CH), :]
        p = [(sp[:, t:t + 1] - qT[t:t + 1, :]) ** 2 for t in range(4)]
        dch = (p[0] + p[1]) + (p[2] + p[3])
        dch = jnp.where(bp == qbT, dch, inf)
        lio = iota(c)
        cd = list(ms)
        ci = list(js)
        for t in range(_K):
            mc = jnp.min(dch, axis=0, keepdims=True)
            jc = jnp.min(jnp.where(dch <= mc, lio, bigi), axis=0, keepdims=True)
            cd.append(mc)
            ci.append(jc)
            if t < _K - 1:
                dch = jnp.where(lio == jc, inf, dch)
        nm, nj = [], []
        for t in range(_K):
            bd, bi = cd[0], ci[0]
            for u in range(1, len(cd)):
                c2 = (cd[u] < bd) | ((cd[u] == bd) & (ci[u] < bi))
                bd = jnp.where(c2, cd[u], bd)
                bi = jnp.where(c2, ci[u], bi)
            nm.append(bd)
            nj.append(bi)
            if t < _K - 1:
                for u in range(len(cd)):
                    sel = ci[u] == bi
                    cd[u] = jnp.where(sel, inf, cd[u])
                    ci[u] = jnp.where(sel, bigi, ci[u])
        return tuple(nm), tuple(nj)

    init = (tuple(jnp.full((1, _QT), inf) for _ in range(_K)),
            tuple(jnp.full((1, _QT), bigi, jnp.int32) for _ in range(_K)))
    ms, js = jax.lax.fori_loop(0, nch, body0, init)

    def bodyg(c, accs):
        k0 = lo + c * _CH
        hc = hc_ref[pl.ds(k0, _CH), :]
        lio = iota(c)
        out = []
        for jt, acc in zip(js, accs):
            oh = (lio == jt).astype(jnp.bfloat16)
            out.append(acc + _dot0b(oh, hc))
        return tuple(out)

    zero = jnp.zeros((_QT, 44), jnp.float32)
    accs = jax.lax.fori_loop(0, nch, bodyg, (zero,) * _K)

    acc_s = acc_m = None
    for m, acc in zip(ms, accs):
        w = jnp.exp(-10.0 * m).T
        msg = (acc[:, :22] + acc[:, 22:]) * w
        acc_s = msg if acc_s is None else acc_s + msg
        acc_m = msg if acc_m is None else jnp.maximum(acc_m, msg)
    agg_ref[...] = jnp.concatenate([acc_s * (1.0 / _K), acc_m], axis=1)


def _knn_call(bounds, s_pad, bcol_pad, hc_pad):
    full4 = pl.BlockSpec((_NP, 4), lambda i: (0, 0))
    full1 = pl.BlockSpec((_NP, 1), lambda i: (0, 0))
    full44 = pl.BlockSpec((_NP, 44), lambda i: (0, 0))
    return pl.pallas_call(
        _knn_body,
        grid=(_NPQ // _QT,),
        in_specs=[
            pl.BlockSpec(memory_space=pltpu.SMEM),
            pl.BlockSpec((_QT, 4), lambda i: (i, 0)),
            pl.BlockSpec((_QT, 1), lambda i: (i, 0)),
            full4, full1, full44,
        ],
        out_specs=pl.BlockSpec((_QT, 44), lambda i: (i, 0)),
        out_shape=jax.ShapeDtypeStruct((_NPQ, 44), jnp.float32),
    )(bounds, s_pad, bcol_pad, s_pad, bcol_pad, hc_pad)


def _block_mlp(x, agg, wo, bo, g1, be1, wf1, bf1, g2, be2, wf2, bf2):
    y = _dotL(jnp.concatenate([x, agg], axis=1), wo[...]) + bo[...]
    y = _bn(y, g1[...], be1[...])
    y = jax.nn.relu(_dotL(y, wf1[...]) + bf1[...])
    y = _bn(y, g2[...], be2[...])
    return _dotL(y, wf2[...]) + bf2[...]


def _mlp_body(x_ref, agg_ref, wo, bo, g1, be1, wf1, bf1, g2, be2, wf2, bf2,
              xo_ref):
    xo_ref[...] = _block_mlp(x_ref[...], agg_ref[...], wo, bo, g1, be1,
                             wf1, bf1, g2, be2, wf2, bf2)


def _mlp_hs_body(x_ref, agg_ref, wo, bo, g1, be1, wf1, bf1, g2, be2, wf2, bf2,
                 wh, bh, ws, bs, xo_ref, s_ref, hc_ref):
    y = _block_mlp(x_ref[...], agg_ref[...], wo, bo, g1, be1,
                   wf1, bf1, g2, be2, wf2, bf2)
    xo_ref[...] = y
    s_ref[...] = _dotL(y, ws[...]) + bs[...]
    hc_ref[...] = _hsplit(_dotL(y, wh[...]) + bh[...])


def _post_body(x0, x1, x2, x3,
               pw0, pb0, pg0, pbe0, pw1, pb1, pg1, pbe1,
               pw2, pb2, pg2, pbe2, pw3, pb3, pg3, pbe3,
               ow1, ob1, ow2, ob2, ow3, ob3, y_ref):
    y = jnp.concatenate([x0[...], x1[...], x2[...], x3[...]], axis=1)
    for (w, b, g, be) in ((pw0, pb0, pg0, pbe0), (pw1, pb1, pg1, pbe1),
                          (pw2, pb2, pg2, pbe2), (pw3, pb3, pg3, pbe3)):
        y = _bn(jax.nn.relu(_dotL(y, w[...]) + b[...]), g[...], be[...])
    y = jax.nn.relu(_dotL(y, ow1[...]) + ob1[...])
    y = jax.nn.relu(_dotL(y, ow2[...]) + ob2[...])
    y_ref[...] = jax.nn.sigmoid(_dotL(y, ow3[...]) + ob3[...])


def _tile_bounds(b32):
    starts = jnp.searchsorted(b32, jnp.arange(_G + 1, dtype=jnp.int32))
    bq = jnp.concatenate(
        [b32, jnp.full((_NPQ - _N,), b32[-1], jnp.int32)])
    bfirst = bq[0::_QT]
    blast = bq[_QT - 1::_QT]
    lo = starts[bfirst].astype(jnp.int32)
    hi = starts[blast + 1].astype(jnp.int32)
    lo = (lo // _CH) * _CH
    nch = (hi - lo + _CH - 1) // _CH
    return jnp.stack([lo, nch], axis=1).astype(jnp.int32)


def kernel(x, batch, params):
    b32 = batch.astype(jnp.int32)
    bcol = b32.reshape(_N, 1)
    bounds = _tile_bounds(b32)
    bcol_pad = jnp.pad(bcol, ((0, _NP - _N), (0, 0)), constant_values=-1)

    def r2(a):
        return a.reshape(1, -1)

    def padk(a):
        return jnp.pad(a, ((0, _NP - _N), (0, 0)))

    blk = params["blocks"]
    pre_w = []
    for p in params["pre"]:
        pre_w += [p["lin"]["w"], r2(p["lin"]["b"]),
                  r2(p["bn"]["g"]), r2(p["bn"]["b"])]
    b0 = blk[0]
    xk, s, hc = pl.pallas_call(
        _pre_body,
        out_shape=[
            jax.ShapeDtypeStruct((_N, 64), jnp.float32),
            jax.ShapeDtypeStruct((_N, 4), jnp.float32),
            jax.ShapeDtypeStruct((_N, 44), jnp.bfloat16),
        ],
    )(x, bcol, *pre_w,
      b0["lin_h"]["w"], r2(b0["lin_h"]["b"]),
      b0["lin_s"]["w"], r2(b0["lin_s"]["b"]))

    outs = []
    for k in range(4):
        agg = _knn_call(bounds, padk(s), bcol_pad, padk(hc))[:_N]
        bk = blk[k]
        mlp_w = [bk["lin_out"]["w"], r2(bk["lin_out"]["b"]),
                 r2(bk["bn1"]["g"]), r2(bk["bn1"]["b"]),
                 bk["fc1"]["w"], r2(bk["fc1"]["b"]),
                 r2(bk["bn2"]["g"]), r2(bk["bn2"]["b"]),
                 bk["fc2"]["w"], r2(bk["fc2"]["b"])]
        if k < 3:
            nb = blk[k + 1]
            xk, s, hc = pl.pallas_call(
                _mlp_hs_body,
                out_shape=[
                    jax.ShapeDtypeStruct((_N, 64), jnp.float32),
                    jax.ShapeDtypeStruct((_N, 4), jnp.float32),
                    jax.ShapeDtypeStruct((_N, 44), jnp.bfloat16),
                ],
            )(xk, agg, *mlp_w,
              nb["lin_h"]["w"], r2(nb["lin_h"]["b"]),
              nb["lin_s"]["w"], r2(nb["lin_s"]["b"]))
        else:
            xk = pl.pallas_call(
                _mlp_body,
                out_shape=jax.ShapeDtypeStruct((_N, 64), jnp.float32),
            )(xk, agg, *mlp_w)
        outs.append(xk)

    post_w = []
    for p in params["post"]:
        post_w += [p["lin"]["w"], r2(p["lin"]["b"]),
                   r2(p["bn"]["g"]), r2(p["bn"]["b"])]
    y = pl.pallas_call(
        _post_body,
        out_shape=jax.ShapeDtypeStruct((_N, 1), jnp.float32),
    )(*outs, *post_w,
      params["out1"]["w"], r2(params["out1"]["b"]),
      params["out2"]["w"], r2(params["out2"]["b"]),
      params["out3"]["w"], r2(params["out3"]["b"]))
    return y
